# baseline (speedup 1.0000x reference)
import functools

import jax
import jax.numpy as jnp
from jax import lax
from jax.experimental import pallas as pl
from jax.experimental.pallas import tpu as pltpu
from jax.experimental.pallas import tpu_sc as plsc

B = 16384
DIM = 64
NUM_FEAT = 16
OUT_D = DIM + NUM_FEAT

NC = 2
NS = 16
NW = NC * NS
BPW = B // NW
L = 16
BLK = 512
TAIL_B = 999936 // BLK
RB = 128


@functools.partial(
    pl.kernel,
    out_type=jax.ShapeDtypeStruct((B, 128), jnp.float32),
    mesh=plsc.VectorSubcoreMesh(core_axis_name="c", subcore_axis_name="s"),
    compiler_params=pltpu.CompilerParams(needs_layout_passes=False),
    scratch_types=[
        pltpu.VMEM((BPW,), jnp.int32),
        pltpu.VMEM((BPW,), jnp.int32),
        pltpu.VMEM((DIM, BLK), jnp.float32),
        pltpu.VMEM((RB, 128), jnp.float32),
        pltpu.SemaphoreType.DMA,
    ],
)
def _sc_sorted_gather(sidx_hbm, spos_hbm, tabt_hbm, tailt_hbm, scat_hbm,
                      sidx_v, spos_v, blk_v, rows_v, ssem):
    wid = lax.axis_index("s") * NC + lax.axis_index("c")
    base = wid * BPW

    pltpu.sync_copy(sidx_hbm.at[pl.ds(base, BPW)], sidx_v)
    pltpu.sync_copy(spos_hbm.at[pl.ds(base, BPW)], spos_v)

    iota = lax.iota(jnp.int32, L)

    def tchunk(t, cur_b):
        v = sidx_v[pl.ds(t * L, L)]
        vb = lax.shift_right_logical(v, 9)
        vq = v & (BLK - 1)
        for j in range(L):
            b = vb[j]

            @pl.when(b != cur_b)
            def _():
                @pl.when(b <= TAIL_B - 1)
                def _():
                    off = pl.multiple_of(b * BLK, 128)
                    pltpu.sync_copy(tabt_hbm.at[:, pl.ds(off, BLK)], blk_v)

                @pl.when(b >= TAIL_B)
                def _():
                    pltpu.sync_copy(tailt_hbm, blk_v.at[:, pl.ds(0, 128)])

            cur_b = b
            q16 = lax.broadcast(vq[j], (L,))
            r = (t * L + j) & (RB - 1)
            for k in range(DIM // L):
                col = plsc.load_gather(blk_v, [iota + (k * L), q16])
                rows_v[r, pl.ds(k * L, L)] = col
        return cur_b

    cur_b = jnp.int32(-1)
    for bt in range(BPW // RB):
        cur_b = lax.fori_loop(bt * (RB // L), (bt + 1) * (RB // L),
                              tchunk, cur_b)
        pltpu.async_copy(
            rows_v,
            scat_hbm.at[spos_v.at[pl.ds(bt * RB, RB)]], ssem).wait()


@functools.partial(
    pl.kernel,
    out_type=jax.ShapeDtypeStruct((OUT_D, B), jnp.float32),
    mesh=plsc.VectorSubcoreMesh(core_axis_name="c", subcore_axis_name="s"),
    compiler_params=pltpu.CompilerParams(needs_layout_passes=False),
    scratch_types=[
        pltpu.VMEM((BPW, 128), jnp.float32),
        pltpu.VMEM((OUT_D, BPW), jnp.float32),
        pltpu.SemaphoreType.DMA,
    ],
)
def _sc_assemble(scat_hbm, numt_hbm, outt_hbm, scat_v, out_v, nsem):
    wid = lax.axis_index("s") * NC + lax.axis_index("c")
    base = wid * BPW

    ncopy = pltpu.async_copy(
        numt_hbm.at[:, pl.ds(base, BPW)],
        out_v.at[pl.ds(DIM, NUM_FEAT)], nsem)
    pltpu.sync_copy(scat_hbm.at[pl.ds(base, BPW)], scat_v)

    iota = lax.iota(jnp.int32, L)

    def group(t, carry):
        i0 = t * L
        i16 = iota + i0
        for c in range(DIM):
            row16 = plsc.load_gather(
                scat_v, [i16, lax.broadcast(jnp.int32(c), (L,))])
            out_v[c, pl.ds(i0, L)] = row16
        return carry

    lax.fori_loop(0, BPW // L, group, 0)

    ncopy.wait()
    pltpu.sync_copy(out_v, outt_hbm.at[:, pl.ds(base, BPW)])


def kernel(c_emb_input, c_numeric, emb_table):
    idx = c_emb_input.astype(jnp.int32)
    spos = jnp.argsort(idx).astype(jnp.int32)
    sidx = idx[spos]
    tabt = emb_table.T
    tailt = jnp.pad(tabt[:, TAIL_B * BLK:1000000], ((0, 0), (0, 64)))
    scat = _sc_sorted_gather(sidx, spos, tabt, tailt)
    outt = _sc_assemble(scat, c_numeric.T)
    return outt.T


# speedup vs baseline: 1.1821x; 1.1821x over previous
import functools

import jax
import jax.numpy as jnp
from jax import lax
from jax.experimental import pallas as pl
from jax.experimental.pallas import tpu as pltpu
from jax.experimental.pallas import tpu_sc as plsc

B = 16384
N_TAB = 1000001
DIM = 64
NUM_FEAT = 16
OUT_D = DIM + NUM_FEAT

NC = 2
NS = 16
NW = NC * NS
BPW = B // NW
L = 16
K = 8
TAIL_C = 999936 // 128


@functools.partial(
    pl.kernel,
    out_type=jax.ShapeDtypeStruct((OUT_D, B), jnp.float32),
    mesh=plsc.VectorSubcoreMesh(core_axis_name="c", subcore_axis_name="s"),
    compiler_params=pltpu.CompilerParams(needs_layout_passes=False),
    scratch_types=[
        pltpu.VMEM((BPW + L,), jnp.int32),
        pltpu.VMEM((K, DIM, 128), jnp.float32),
        pltpu.VMEM((OUT_D, BPW), jnp.float32),
        pltpu.SemaphoreType.DMA,
        pltpu.SemaphoreType.DMA,
    ],
)
def _sc_panel_gather(idx_hbm, numt_hbm, tabt_hbm, tailt_hbm, outt_hbm,
                     idx_v, panel_v, out_v, gsem, nsem):
    wid = lax.axis_index("s") * NC + lax.axis_index("c")
    base = wid * BPW

    pltpu.sync_copy(idx_hbm.at[pl.ds(base, BPW)], idx_v.at[pl.ds(0, BPW)])
    ncopy = pltpu.async_copy(
        numt_hbm.at[:, pl.ds(base, BPW)],
        out_v.at[pl.ds(DIM, NUM_FEAT)], nsem)

    iota = lax.iota(jnp.int32, L)

    def fire(cs, j):
        @pl.when(cs <= TAIL_C - 1)
        def _():
            off = pl.multiple_of(cs * 128, 128)
            pltpu.async_copy(
                tabt_hbm.at[:, pl.ds(off, 128)], panel_v.at[j], gsem)

        @pl.when(cs >= TAIL_C)
        def _():
            pltpu.async_copy(tailt_hbm, panel_v.at[j], gsem)

    v0 = idx_v[pl.ds(0, L)]
    vc0 = lax.shift_right_logical(v0, 7)
    for j in range(K):
        fire(vc0[j], j)

    def ring(it, carry):
        i0 = it * K
        v = idx_v[pl.ds(i0, L)]
        vc = lax.shift_right_logical(v, 7)
        vq = v & 127
        for j in range(K):
            pltpu.make_async_copy(tailt_hbm, panel_v.at[j], gsem).wait()
            q16 = lax.broadcast(vq[j], (L,))
            i16 = lax.broadcast(i0 + j, (L,))
            for k in range(DIM // L):
                col = plsc.load_gather(
                    panel_v, [lax.broadcast(j, (L,)), iota + (k * L), q16])
                plsc.store_scatter(out_v, [iota + (k * L), i16], col)

            @pl.when(i0 + K + j < BPW)
            def _():
                fire(vc[K + j], j)
        return carry

    lax.fori_loop(0, BPW // K, ring, 0)

    ncopy.wait()
    pltpu.sync_copy(out_v, outt_hbm.at[:, pl.ds(base, BPW)])


def kernel(c_emb_input, c_numeric, emb_table):
    idx = c_emb_input.astype(jnp.int32)
    tabt = emb_table.T
    tailt = jnp.pad(tabt[:, TAIL_C * 128:1000000], ((0, 0), (0, 64)))
    outt = _sc_panel_gather(idx, c_numeric.T, tabt, tailt)
    return outt.T
